# Initial kernel scaffold; baseline (speedup 1.0000x reference)
import jax
import jax.numpy as jnp
from jax.experimental import pallas as pl


def kernel(ent_feat, edge_index, W_ent, attn_h, attn_t, ln1_g, ln1_b, ln2_g, ln2_b, W_ff1, b_ff1, W_ff2, b_ff2):
    raise NotImplementedError("write your pallas kernel here")


# speedup vs baseline: 27.5482x; 27.5482x over previous
import jax
import jax.numpy as jnp
from jax import lax
from jax.experimental import pallas as pl
from jax.experimental.pallas import tpu as pltpu
from jax.experimental.pallas import tpu_sc as plsc

N = 10000
E = 320000
D = 128
H = 8
DH = 16
ALPHA = 0.15
HOPS = 5
NEG = 0.2
DFF = 4 * D

NC = 2
NS = 16
NG = 4
FW = D // NG
EPT = E // NS
CHUNK = 80
NCHUNK = EPT // CHUNK
NPAD = 10240
NPT = NPAD // NS
HPC = H // NC
BM = 1000


def _head_body(x_ref, wt_ref, ah_ref, at_ref, g_ref, b_ref,
               h_ref, feat_ref, tbl_ref):
    x = x_ref[...]
    mu = jnp.mean(x, axis=1, keepdims=True)
    xc = x - mu
    var = jnp.mean(xc * xc, axis=1, keepdims=True)
    h = xc * lax.rsqrt(var + 1e-5) * g_ref[...] + b_ref[...]
    feat = jnp.dot(h, wt_ref[...], preferred_element_type=jnp.float32)
    eh = jnp.dot(feat, ah_ref[...], preferred_element_type=jnp.float32)
    et = jnp.dot(feat, at_ref[...], preferred_element_type=jnp.float32)
    h_ref[...] = h
    feat_ref[...] = feat
    tbl_ref[...] = jnp.concatenate([eh, et], axis=1)


def _head(x, wt, ah, at, g, b):
    grid = N // BM
    return pl.pallas_call(
        _head_body,
        grid=(grid,),
        in_specs=[
            pl.BlockSpec((BM, D), lambda i: (i, 0)),
            pl.BlockSpec((D, D), lambda i: (0, 0)),
            pl.BlockSpec((D, H), lambda i: (0, 0)),
            pl.BlockSpec((D, H), lambda i: (0, 0)),
            pl.BlockSpec((1, D), lambda i: (0, 0)),
            pl.BlockSpec((1, D), lambda i: (0, 0)),
        ],
        out_specs=[
            pl.BlockSpec((BM, D), lambda i: (i, 0)),
            pl.BlockSpec((BM, D), lambda i: (i, 0)),
            pl.BlockSpec((BM, 2 * H), lambda i: (i, 0)),
        ],
        out_shape=[
            jax.ShapeDtypeStruct((N, D), jnp.float32),
            jax.ShapeDtypeStruct((N, D), jnp.float32),
            jax.ShapeDtypeStruct((N, 2 * H), jnp.float32),
        ],
    )(x, wt, ah, at, g, b)


def _ffn_body(h_ref, f0_ref, f1_ref, f2_ref, f3_ref, g_ref, b_ref,
              w1t_ref, b1_ref, w2t_ref, b2_ref, o_ref):
    f = jnp.concatenate(
        [f0_ref[0], f1_ref[0], f2_ref[0], f3_ref[0]], axis=1)
    rst = f + h_ref[...]
    mu = jnp.mean(rst, axis=1, keepdims=True)
    xc = rst - mu
    var = jnp.mean(xc * xc, axis=1, keepdims=True)
    x2 = xc * lax.rsqrt(var + 1e-5) * g_ref[...] + b_ref[...]
    y = jnp.dot(x2, w1t_ref[...], preferred_element_type=jnp.float32)
    y = jnp.maximum(y + b1_ref[...], 0.0)
    ff = jnp.dot(y, w2t_ref[...], preferred_element_type=jnp.float32)
    o_ref[...] = ff + b2_ref[...] + rst


def _ffn(h, fsplit, g, b, w1t, b1, w2t, b2):
    grid = N // BM

    def fspec(gq):
        return pl.BlockSpec((1, BM, FW), lambda i, gq=gq: (gq, i, 0))

    return pl.pallas_call(
        _ffn_body,
        grid=(grid,),
        in_specs=[
            pl.BlockSpec((BM, D), lambda i: (i, 0)),
            fspec(0), fspec(1), fspec(2), fspec(3),
            pl.BlockSpec((1, D), lambda i: (0, 0)),
            pl.BlockSpec((1, D), lambda i: (0, 0)),
            pl.BlockSpec((D, DFF), lambda i: (0, 0)),
            pl.BlockSpec((1, DFF), lambda i: (0, 0)),
            pl.BlockSpec((DFF, D), lambda i: (0, 0)),
            pl.BlockSpec((1, D), lambda i: (0, 0)),
        ],
        out_specs=[pl.BlockSpec((BM, D), lambda i: (i, 0))],
        out_shape=[jax.ShapeDtypeStruct((N, D), jnp.float32)],
    )(h, fsplit, fsplit, fsplit, fsplit, g, b, w1t, b1, w2t, b2)[0]


def _dyn_gather(v, idx):
    dnums = lax.GatherDimensionNumbers(
        offset_dims=(), collapsed_slice_dims=(0,), start_index_map=(0,))
    return lax.gather(v, idx[:, None], dnums, slice_sizes=(1,),
                      mode=lax.GatherScatterMode.PROMISE_IN_BOUNDS)


def _sc_body(tbl, srcv, dstv, f0q, fout,
             src_b, dst_b, hs_b, td_b, ex_b, es_b, rows_b, m_b,
             cmb_a, cmb_f, z16_b, tbl_sh, esum_sh, fa_sh, fb_sh,
             a_h, sem):
    c = lax.axis_index("c")
    s = lax.axis_index("s")
    e_base = s * EPT
    n0 = s * NPT
    lane = lax.iota(jnp.int32, 16)
    h0 = HPC * c
    mask = (lane >= h0) & (lane < h0 + HPC)
    cE = c * E
    zero16 = jnp.zeros((16,), jnp.float32)
    shift8 = (lane & 7) + 8

    @pl.when(s < NS - 1)
    def _stage_full():
        pltpu.sync_copy(tbl.at[pl.ds(n0, NPT)], tbl_sh.at[pl.ds(n0, NPT)])

    @pl.when(s == NS - 1)
    def _stage_tail():
        pltpu.sync_copy(tbl.at[pl.ds(N - 400, 400)],
                        tbl_sh.at[pl.ds(N - 400, 400)])

    @pl.loop(0, NPT)
    def _z0(r):
        z16_b[r, :] = zero16

    pltpu.sync_copy(z16_b, esum_sh.at[pl.ds(n0, NPT)])
    plsc.subcore_barrier()

    @pl.loop(0, NCHUNK)
    def _p1(j):
        e0 = e_base + j * CHUNK
        pltpu.sync_copy(srcv.at[pl.ds(e0, CHUNK)], src_b)
        pltpu.sync_copy(dstv.at[pl.ds(e0, CHUNK)], dst_b)
        pltpu.async_copy(tbl_sh.at[src_b], hs_b, sem).wait()
        pltpu.async_copy(tbl_sh.at[dst_b], td_b, sem).wait()

        @pl.loop(0, CHUNK)
        def _e(i):
            e16 = hs_b[i, :] + _dyn_gather(td_b[i, :], shift8)
            e16 = jnp.maximum(e16, NEG * e16)
            ex_b[i, :] = jnp.where(mask, jnp.exp(e16), 0.0)

        pltpu.sync_copy(ex_b, esum_sh.at[dst_b], add=True)

    plsc.subcore_barrier()

    @pl.loop(0, NCHUNK)
    def _p2(j):
        e0 = e_base + j * CHUNK
        pltpu.sync_copy(srcv.at[pl.ds(e0, CHUNK)], src_b)
        pltpu.sync_copy(dstv.at[pl.ds(e0, CHUNK)], dst_b)
        pltpu.async_copy(tbl_sh.at[src_b], hs_b, sem).wait()
        pltpu.async_copy(tbl_sh.at[dst_b], td_b, sem).wait()
        pltpu.async_copy(esum_sh.at[dst_b], es_b, sem).wait()

        @pl.loop(0, CHUNK)
        def _e(i):
            e16 = hs_b[i, :] + _dyn_gather(td_b[i, :], shift8)
            e16 = jnp.maximum(e16, NEG * e16)
            ex = jnp.where(mask, jnp.exp(e16), 0.0)
            ex_b[i, :] = ex / (es_b[i, :] + 1e-16)

        pltpu.sync_copy(ex_b, a_h.at[pl.ds(cE + e0, CHUNK)])

    plsc.subcore_barrier()

    def hop(p, fin_sh, facc_sh):
        @pl.loop(0, NPT)
        def _za(r):
            for q in range(FW // 16):
                cmb_a[r, pl.ds(q * 16, 16)] = zero16

        pltpu.sync_copy(cmb_a, facc_sh.at[pl.ds(n0, NPT)])
        plsc.subcore_barrier()

        @pl.loop(0, NCHUNK)
        def _ch(j):
            e0 = e_base + j * CHUNK
            pltpu.sync_copy(srcv.at[pl.ds(e0, CHUNK)], src_b)
            pltpu.sync_copy(dstv.at[pl.ds(e0, CHUNK)], dst_b)
            pltpu.async_copy(fin_sh.at[src_b], rows_b, sem).wait()
            pltpu.sync_copy(a_h.at[pl.ds(cE + e0, CHUNK)], hs_b)

            @pl.loop(0, CHUNK)
            def _e(i):
                av = hs_b[i, :]
                for hh in range(2):
                    cf = _dyn_gather(
                        av, jnp.full((16,), h0 + 2 * p + hh, jnp.int32))
                    sl = pl.ds(hh * 16, 16)
                    m_b[i, sl] = rows_b[i, sl] * cf

            pltpu.sync_copy(m_b, facc_sh.at[dst_b], add=True)

        plsc.subcore_barrier()

        pltpu.sync_copy(facc_sh.at[pl.ds(n0, NPT)], cmb_a)
        gq = 2 * c + p
        pltpu.sync_copy(f0q.at[pl.ds(gq * NPAD + n0, NPT)], cmb_f)

        @pl.loop(0, NPT)
        def _cm(r):
            for q in range(FW // 16):
                sl = pl.ds(q * 16, 16)
                cmb_a[r, sl] = (1.0 - ALPHA) * cmb_a[r, sl] + ALPHA * cmb_f[r, sl]

        pltpu.sync_copy(cmb_a, facc_sh.at[pl.ds(n0, NPT)])
        plsc.subcore_barrier()

    for p in range(2):
        base = (2 * c + p) * NPAD
        pltpu.sync_copy(f0q.at[pl.ds(base + n0, NPT)],
                        fa_sh.at[pl.ds(n0, NPT)])
        plsc.subcore_barrier()
        hop(p, fa_sh, fb_sh)
        hop(p, fb_sh, fa_sh)
        hop(p, fa_sh, fb_sh)
        hop(p, fb_sh, fa_sh)
        hop(p, fa_sh, fb_sh)
        pltpu.sync_copy(fb_sh.at[pl.ds(n0, NPT)],
                        fout.at[pl.ds(base + n0, NPT)])
        plsc.subcore_barrier()


def _sc_diffuse(tbl, src, dst, f0q):
    mesh = plsc.VectorSubcoreMesh(core_axis_name="c", subcore_axis_name="s")
    fn = pl.kernel(
        _sc_body,
        out_type=jax.ShapeDtypeStruct((NG * NPAD, FW), jnp.float32),
        mesh=mesh,
        compiler_params=pltpu.CompilerParams(use_tc_tiling_on_sc=False),
        scratch_types=[
            pltpu.VMEM((CHUNK,), jnp.int32),
            pltpu.VMEM((CHUNK,), jnp.int32),
            pltpu.VMEM((CHUNK, 16), jnp.float32),
            pltpu.VMEM((CHUNK, 16), jnp.float32),
            pltpu.VMEM((CHUNK, 16), jnp.float32),
            pltpu.VMEM((CHUNK, 16), jnp.float32),
            pltpu.VMEM((CHUNK, FW), jnp.float32),
            pltpu.VMEM((CHUNK, FW), jnp.float32),
            pltpu.VMEM((NPT, FW), jnp.float32),
            pltpu.VMEM((NPT, FW), jnp.float32),
            pltpu.VMEM((NPT, 16), jnp.float32),
            pltpu.VMEM_SHARED((NPAD, 16), jnp.float32),
            pltpu.VMEM_SHARED((NPAD, 16), jnp.float32),
            pltpu.VMEM_SHARED((NPAD, FW), jnp.float32),
            pltpu.VMEM_SHARED((NPAD, FW), jnp.float32),
            pltpu.HBM((NC * E, 16), jnp.float32),
            pltpu.SemaphoreType.DMA,
        ],
    )
    return fn(tbl, src, dst, f0q)


def kernel(ent_feat, edge_index, W_ent, attn_h, attn_t, ln1_g, ln1_b,
           ln2_g, ln2_b, W_ff1, b_ff1, W_ff2, b_ff2):
    src = edge_index[0].astype(jnp.int32)
    dst = edge_index[1].astype(jnp.int32)
    eye = jnp.eye(H, dtype=jnp.float32)
    ah = (attn_h[0][:, :, None] * eye[:, None, :]).reshape(D, H)
    at = (attn_t[0][:, :, None] * eye[:, None, :]).reshape(D, H)

    h, feat, tbl = _head(ent_feat, W_ent.T, ah, at,
                         ln1_g.reshape(1, D), ln1_b.reshape(1, D))
    zpad = jnp.zeros((NPAD - N, FW), jnp.float32)
    f0q = jnp.concatenate(
        [x for g in range(NG) for x in (feat[:, g * FW:(g + 1) * FW], zpad)],
        axis=0)
    fsplit = _sc_diffuse(tbl, src, dst, f0q).reshape(NG, NPAD, FW)
    out = _ffn(h, fsplit, ln2_g.reshape(1, D), ln2_b.reshape(1, D),
               W_ff1.T, b_ff1.reshape(1, DFF), W_ff2.T, b_ff2.reshape(1, D))
    return out


# speedup vs baseline: 40.6502x; 1.4756x over previous
import jax
import jax.numpy as jnp
from jax import lax
from jax.experimental import pallas as pl
from jax.experimental.pallas import tpu as pltpu
from jax.experimental.pallas import tpu_sc as plsc

N = 10000
E = 320000
D = 128
H = 8
DH = 16
ALPHA = 0.15
HOPS = 5
NEG = 0.2
DFF = 4 * D

NC = 2
NS = 16
NG = 4
FW = D // NG
EPT = E // NS
CHUNK = 80
NCHUNK = EPT // CHUNK
NPAD = 10240
NPT = NPAD // NS
HPC = H // NC
BM = 1000


def _head_body(x_ref, wt_ref, ah_ref, at_ref, g_ref, b_ref,
               h_ref, feat_ref, tbl_ref):
    x = x_ref[...]
    mu = jnp.mean(x, axis=1, keepdims=True)
    xc = x - mu
    var = jnp.mean(xc * xc, axis=1, keepdims=True)
    h = xc * lax.rsqrt(var + 1e-5) * g_ref[...] + b_ref[...]
    feat = jnp.dot(h, wt_ref[...], preferred_element_type=jnp.float32)
    eh = jnp.dot(feat, ah_ref[...], preferred_element_type=jnp.float32)
    et = jnp.dot(feat, at_ref[...], preferred_element_type=jnp.float32)
    h_ref[...] = h
    feat_ref[...] = feat
    zp = jnp.zeros_like(eh)
    tbl_ref[...] = jnp.concatenate([eh, et, zp, zp], axis=1)


def _head(x, wt, ah, at, g, b):
    grid = N // BM
    return pl.pallas_call(
        _head_body,
        grid=(grid,),
        in_specs=[
            pl.BlockSpec((BM, D), lambda i: (i, 0)),
            pl.BlockSpec((D, D), lambda i: (0, 0)),
            pl.BlockSpec((D, H), lambda i: (0, 0)),
            pl.BlockSpec((D, H), lambda i: (0, 0)),
            pl.BlockSpec((1, D), lambda i: (0, 0)),
            pl.BlockSpec((1, D), lambda i: (0, 0)),
        ],
        out_specs=[
            pl.BlockSpec((BM, D), lambda i: (i, 0)),
            pl.BlockSpec((BM, D), lambda i: (i, 0)),
            pl.BlockSpec((BM, 2 * FW // 2), lambda i: (i, 0)),
        ],
        out_shape=[
            jax.ShapeDtypeStruct((N, D), jnp.float32),
            jax.ShapeDtypeStruct((N, D), jnp.float32),
            jax.ShapeDtypeStruct((N, FW), jnp.float32),
        ],
    )(x, wt, ah, at, g, b)


def _ffn_body(h_ref, f0_ref, f1_ref, f2_ref, f3_ref, g_ref, b_ref,
              w1t_ref, b1_ref, w2t_ref, b2_ref, o_ref):
    f = jnp.concatenate(
        [f0_ref[0], f1_ref[0], f2_ref[0], f3_ref[0]], axis=1)
    rst = f + h_ref[...]
    mu = jnp.mean(rst, axis=1, keepdims=True)
    xc = rst - mu
    var = jnp.mean(xc * xc, axis=1, keepdims=True)
    x2 = xc * lax.rsqrt(var + 1e-5) * g_ref[...] + b_ref[...]
    y = jnp.dot(x2, w1t_ref[...], preferred_element_type=jnp.float32)
    y = jnp.maximum(y + b1_ref[...], 0.0)
    ff = jnp.dot(y, w2t_ref[...], preferred_element_type=jnp.float32)
    o_ref[...] = ff + b2_ref[...] + rst


def _ffn(h, fsplit, g, b, w1t, b1, w2t, b2):
    grid = N // BM

    def fspec(gq):
        return pl.BlockSpec((1, BM, FW), lambda i, gq=gq: (gq, i, 0))

    return pl.pallas_call(
        _ffn_body,
        grid=(grid,),
        in_specs=[
            pl.BlockSpec((BM, D), lambda i: (i, 0)),
            fspec(0), fspec(1), fspec(2), fspec(3),
            pl.BlockSpec((1, D), lambda i: (0, 0)),
            pl.BlockSpec((1, D), lambda i: (0, 0)),
            pl.BlockSpec((D, DFF), lambda i: (0, 0)),
            pl.BlockSpec((1, DFF), lambda i: (0, 0)),
            pl.BlockSpec((DFF, D), lambda i: (0, 0)),
            pl.BlockSpec((1, D), lambda i: (0, 0)),
        ],
        out_specs=[pl.BlockSpec((BM, D), lambda i: (i, 0))],
        out_shape=[jax.ShapeDtypeStruct((N, D), jnp.float32)],
    )(h, fsplit, fsplit, fsplit, fsplit, g, b, w1t, b1, w2t, b2)[0]


def _dyn_gather(v, idx):
    dnums = lax.GatherDimensionNumbers(
        offset_dims=(), collapsed_slice_dims=(0,), start_index_map=(0,))
    return lax.gather(v, idx[:, None], dnums, slice_sizes=(1,),
                      mode=lax.GatherScatterMode.PROMISE_IN_BOUNDS)


def _sc_body(tbl, srcv, dstv, f0q, fout,
             sidx_v, didx_v, src_b, dg_b, dst_b, hs_b, td_b, ex_b, es_b,
             a16_b, rows_b, m_b, cmb_a, cmb_f, fab_sh,
             a_h, sem, sem2, sem3):
    c = lax.axis_index("c")
    s = lax.axis_index("s")
    e_base = s * EPT
    n0 = s * NPT
    lane = lax.iota(jnp.int32, 16)
    h0 = HPC * c
    mask = (lane >= h0) & (lane < h0 + HPC)
    cE = c * E
    zero16 = jnp.zeros((16,), jnp.float32)
    shift8 = (lane & 7) + 8


    @pl.when(s < NS - 1)
    def _stage_full():
        pltpu.sync_copy(tbl.at[pl.ds(n0, NPT)],
                        fab_sh.at[pl.ds(NPAD + n0, NPT)])

    @pl.when(s == NS - 1)
    def _stage_tail():
        pltpu.sync_copy(tbl.at[pl.ds(N - 400, 400)],
                        fab_sh.at[pl.ds(NPAD + N - 400, 400)])

    @pl.loop(0, NPT)
    def _z1(r):
        for q in range(FW // 16):
            cmb_a[r, pl.ds(q * 16, 16)] = zero16

    pltpu.sync_copy(cmb_a, fab_sh.at[pl.ds(n0, NPT)])

    @pl.loop(0, CHUNK)
    def _z2(i):
        ex_b[i, pl.ds(0, 16)] = zero16
        ex_b[i, pl.ds(16, 16)] = zero16

    plsc.subcore_barrier()

    def load_idx(j, sem_a, sem_b):
        e0 = e_base + j * CHUNK
        return (pltpu.async_copy(srcv.at[pl.ds(e0, CHUNK)], sidx_v, sem_a),
                pltpu.async_copy(dstv.at[pl.ds(e0, CHUNK)], didx_v, sem_b))

    def fill_idx(src_off, dst_off):
        @pl.loop(0, CHUNK // 16)
        def _f(k):
            kl = pl.ds(k * 16, 16)
            src_b[kl] = sidx_v[kl] + src_off
            dst_b[kl] = didx_v[kl] + dst_off

    def fill_dg(dst_off):
        @pl.loop(0, CHUNK // 16)
        def _f(k):
            kl = pl.ds(k * 16, 16)
            dg_b[kl] = didx_v[kl] + dst_off

    @pl.loop(0, NCHUNK)
    def _p1(j):
        e0 = e_base + j * CHUNK
        i1, i2 = load_idx(j, sem, sem2)
        i1.wait()
        i2.wait()
        fill_idx(NPAD, 0)
        fill_dg(NPAD)
        d1 = pltpu.async_copy(fab_sh.at[src_b], hs_b, sem)
        d2 = pltpu.async_copy(fab_sh.at[dg_b], td_b, sem2)
        d1.wait()
        d2.wait()

        @pl.loop(0, CHUNK)
        def _e(i):
            e16 = hs_b[i, pl.ds(0, 16)] + _dyn_gather(
                td_b[i, pl.ds(0, 16)], shift8)
            e16 = jnp.maximum(e16, NEG * e16)
            ex = jnp.where(mask, jnp.exp(e16), 0.0)
            ex_b[i, pl.ds(0, 16)] = ex
            a16_b[i, :] = ex

        pltpu.sync_copy(ex_b, fab_sh.at[dst_b], add=True)
        pltpu.sync_copy(a16_b, a_h.at[pl.ds(cE + e0, CHUNK)])

    plsc.subcore_barrier()

    @pl.loop(0, NCHUNK)
    def _p2(j):
        e0 = e_base + j * CHUNK
        i1, i2 = load_idx(j, sem, sem2)
        d1 = pltpu.async_copy(a_h.at[pl.ds(cE + e0, CHUNK)], a16_b, sem3)
        i1.wait()
        i2.wait()
        fill_idx(0, 0)
        d2 = pltpu.async_copy(fab_sh.at[dst_b], es_b, sem2)
        d1.wait()
        d2.wait()

        @pl.loop(0, CHUNK)
        def _e(i):
            a16_b[i, :] = a16_b[i, :] / (es_b[i, pl.ds(0, 16)] + 1e-16)

        pltpu.sync_copy(a16_b, a_h.at[pl.ds(cE + e0, CHUNK)])

    plsc.subcore_barrier()

    @pl.loop(0, 2)
    def _grp(p):
        base = (2 * c + p) * NPAD
        pltpu.sync_copy(f0q.at[pl.ds(base + n0, NPT)],
                        fab_sh.at[pl.ds(n0, NPT)])
        plsc.subcore_barrier()

        @pl.loop(0, HOPS)
        def _hop(k):
            fin_off = (k & 1) * NPAD
            facc_off = NPAD - fin_off

            @pl.loop(0, NPT)
            def _za(r):
                for q in range(FW // 16):
                    cmb_a[r, pl.ds(q * 16, 16)] = zero16

            pltpu.sync_copy(cmb_a, fab_sh.at[pl.ds(facc_off + n0, NPT)])
            plsc.subcore_barrier()

            @pl.loop(0, NCHUNK)
            def _ch(j):
                e0 = e_base + j * CHUNK
                i1, i2 = load_idx(j, sem, sem2)
                d2 = pltpu.async_copy(
                    a_h.at[pl.ds(cE + e0, CHUNK)], a16_b, sem3)
                i1.wait()
                i2.wait()
                fill_idx(fin_off, facc_off)
                d1 = pltpu.async_copy(fab_sh.at[src_b], rows_b, sem)
                d1.wait()
                d2.wait()

                @pl.loop(0, CHUNK)
                def _e(i):
                    av = a16_b[i, :]
                    for hh in range(2):
                        cf = _dyn_gather(
                            av, jnp.full((16,), h0 + 2 * p + hh, jnp.int32))
                        sl = pl.ds(hh * 16, 16)
                        m_b[i, sl] = rows_b[i, sl] * cf

                pltpu.sync_copy(m_b, fab_sh.at[dst_b], add=True)

            plsc.subcore_barrier()

            pltpu.sync_copy(fab_sh.at[pl.ds(facc_off + n0, NPT)], cmb_a)
            pltpu.sync_copy(f0q.at[pl.ds(base + n0, NPT)], cmb_f)

            @pl.loop(0, NPT)
            def _cm(r):
                for q in range(FW // 16):
                    sl = pl.ds(q * 16, 16)
                    cmb_a[r, sl] = ((1.0 - ALPHA) * cmb_a[r, sl]
                                    + ALPHA * cmb_f[r, sl])

            pltpu.sync_copy(cmb_a, fab_sh.at[pl.ds(facc_off + n0, NPT)])
            plsc.subcore_barrier()

        pltpu.sync_copy(fab_sh.at[pl.ds(NPAD + n0, NPT)],
                        fout.at[pl.ds(base + n0, NPT)])
        plsc.subcore_barrier()


def _sc_diffuse(tbl, src, dst, f0q):
    mesh = plsc.VectorSubcoreMesh(core_axis_name="c", subcore_axis_name="s")
    fn = pl.kernel(
        _sc_body,
        out_type=jax.ShapeDtypeStruct((NG * NPAD, FW), jnp.float32),
        mesh=mesh,
        compiler_params=pltpu.CompilerParams(use_tc_tiling_on_sc=False),
        scratch_types=[
            pltpu.VMEM((CHUNK,), jnp.int32),
            pltpu.VMEM((CHUNK,), jnp.int32),
            pltpu.VMEM((CHUNK,), jnp.int32),
            pltpu.VMEM((CHUNK,), jnp.int32),
            pltpu.VMEM((CHUNK,), jnp.int32),
            pltpu.VMEM((CHUNK, FW), jnp.float32),
            pltpu.VMEM((CHUNK, FW), jnp.float32),
            pltpu.VMEM((CHUNK, FW), jnp.float32),
            pltpu.VMEM((CHUNK, FW), jnp.float32),
            pltpu.VMEM((CHUNK, 16), jnp.float32),
            pltpu.VMEM((CHUNK, FW), jnp.float32),
            pltpu.VMEM((CHUNK, FW), jnp.float32),
            pltpu.VMEM((NPT, FW), jnp.float32),
            pltpu.VMEM((NPT, FW), jnp.float32),
            pltpu.VMEM_SHARED((2 * NPAD, FW), jnp.float32),
            pltpu.HBM((NC * E, 16), jnp.float32),
            pltpu.SemaphoreType.DMA,
            pltpu.SemaphoreType.DMA,
            pltpu.SemaphoreType.DMA,
        ],
    )
    return fn(tbl, src, dst, f0q)


def kernel(ent_feat, edge_index, W_ent, attn_h, attn_t, ln1_g, ln1_b,
           ln2_g, ln2_b, W_ff1, b_ff1, W_ff2, b_ff2):
    src = edge_index[0].astype(jnp.int32)
    dst = edge_index[1].astype(jnp.int32)
    eye = jnp.eye(H, dtype=jnp.float32)
    ah = (attn_h[0][:, :, None] * eye[:, None, :]).reshape(D, H)
    at = (attn_t[0][:, :, None] * eye[:, None, :]).reshape(D, H)

    h, feat, tbl = _head(ent_feat, W_ent.T, ah, at,
                         ln1_g.reshape(1, D), ln1_b.reshape(1, D))
    zpad = jnp.zeros((NPAD - N, FW), jnp.float32)
    f0q = jnp.concatenate(
        [x for g in range(NG) for x in (feat[:, g * FW:(g + 1) * FW], zpad)],
        axis=0)
    fsplit = _sc_diffuse(tbl, src, dst, f0q).reshape(NG, NPAD, FW)
    out = _ffn(h, fsplit, ln2_g.reshape(1, D), ln2_b.reshape(1, D),
               W_ff1.T, b_ff1.reshape(1, DFF), W_ff2.T, b_ff2.reshape(1, D))
    return out
